# baseline (speedup 1.0000x reference)
import functools

import jax
import jax.numpy as jnp
from jax import lax
from jax.experimental import pallas as pl
from jax.experimental.pallas import tpu as pltpu
from jax.experimental.pallas import tpu_sc as plsc

_CHUNK = 80
_NBUF = 2


def _ln_rows(emb, w, b):
    mean = jnp.mean(emb, axis=-1, keepdims=True)
    var = jnp.mean(jnp.square(emb - mean), axis=-1, keepdims=True)
    return (emb - mean) * lax.rsqrt(var + 1e-5) * w + b


def _pair_tbl_body(tok_ref, pos_ref, w_ref, b_ref, tbl_ref):
    vocab, d = tok_ref.shape
    seq = pos_ref.shape[0]
    s2 = seq // 2
    v1 = pl.program_id(0)
    w = w_ref[:][None, None, :]
    b = b_ref[:][None, None, :]
    pos2 = pos_ref[:].reshape(s2, 2, d)
    even = pos2[:, 0, :]
    odd = pos2[:, 1, :]
    rows_even = _ln_rows(
        (tok_ref[pl.ds(v1, 1)] + even)[None], w, b)
    rows_odd = _ln_rows(
        tok_ref[:][:, None, :] + odd[None, :, :], w, b)
    tbl_ref[0] = jnp.concatenate(
        [jnp.broadcast_to(rows_even[0][None], (vocab, s2, d)), rows_odd],
        axis=-1)


def _pair_idx_body(x_ref, idx_ref):
    vocab = 29
    xb = x_ref[...]
    bb, seq = xb.shape
    s2 = seq // 2
    xp = xb.reshape(bb, s2, 2)
    s_iota = lax.broadcasted_iota(jnp.int32, (bb, s2), 1)
    idx_ref[...] = (xp[:, :, 0] * vocab + xp[:, :, 1]) * s2 + s_iota


def _build_pair_table_and_idx(x, tok_embed, pos_embed, ln_w, ln_b):
    vocab, d = tok_embed.shape
    batch, seq = x.shape
    s2 = seq // 2
    tbl = pl.pallas_call(
        _pair_tbl_body,
        grid=(vocab,),
        in_specs=[
            pl.BlockSpec((vocab, d), lambda i: (0, 0)),
            pl.BlockSpec((seq, d), lambda i: (0, 0)),
            pl.BlockSpec((d,), lambda i: (0,)),
            pl.BlockSpec((d,), lambda i: (0,)),
        ],
        out_specs=pl.BlockSpec((1, vocab, s2, 2 * d), lambda i: (i, 0, 0, 0)),
        out_shape=jax.ShapeDtypeStruct((vocab, vocab, s2, 2 * d), jnp.float32),
    )(tok_embed, pos_embed[:seq], ln_w, ln_b)
    n_blocks = 8
    bb = batch // n_blocks
    idx = pl.pallas_call(
        _pair_idx_body,
        grid=(n_blocks,),
        in_specs=[pl.BlockSpec((bb, seq), lambda i: (i, 0))],
        out_specs=pl.BlockSpec((bb, s2), lambda i: (i, 0)),
        out_shape=jax.ShapeDtypeStruct((batch, s2), jnp.int32),
    )(x)
    return tbl.reshape(vocab * vocab * s2, 2 * d), idx.reshape(-1)


def _make_sc_gather(n_rows, d2, n_workers):
    rows_per_w = n_rows // n_workers
    n_chunks = rows_per_w // _CHUNK
    n_groups = (n_chunks - 1) // _NBUF
    n_tail = n_chunks - n_groups * _NBUF
    mesh = plsc.VectorSubcoreMesh(core_axis_name="c", subcore_axis_name="s")

    @functools.partial(
        pl.kernel,
        mesh=mesh,
        out_type=jax.ShapeDtypeStruct((n_rows, d2), jnp.float32),
        scratch_types=[
            pltpu.VMEM((n_chunks, _CHUNK), jnp.int32),
            [pltpu.VMEM((_CHUNK, d2), jnp.float32)] * _NBUF,
            [pltpu.SemaphoreType.DMA] * _NBUF,
            [pltpu.SemaphoreType.DMA] * _NBUF,
        ],
    )
    def sc_gather(tbl_hbm, idx_hbm, out_hbm, idx_v, bufs, gsems, wsems):
        n_cores = lax.axis_size("c")
        wid = lax.axis_index("s") * n_cores + lax.axis_index("c")
        cbase = wid * n_chunks
        pltpu.sync_copy(idx_hbm.at[pl.ds(cbase, n_chunks)], idx_v)
        pltpu.async_copy(tbl_hbm.at[idx_v.at[0]], bufs[0], gsems[0])

        def gather_wait(i, p):
            pltpu.make_async_copy(tbl_hbm.at[idx_v.at[i]], bufs[p], gsems[p]).wait()

        def write_start(i, p):
            pltpu.async_copy(
                bufs[p], out_hbm.at[pl.ds((cbase + i) * _CHUNK, _CHUNK)], wsems[p])

        def write_wait(i, p):
            pltpu.make_async_copy(
                bufs[p], out_hbm.at[pl.ds((cbase + i) * _CHUNK, _CHUNK)], wsems[p]
            ).wait()

        def step(i, p, pn):
            gather_wait(i, p)
            write_start(i, p)
            @pl.when(i >= _NBUF - 1)
            def _():
                write_wait(i - (_NBUF - 1), pn)

            pltpu.async_copy(tbl_hbm.at[idx_v.at[i + 1]], bufs[pn], gsems[pn])

        def body(j, carry):
            i0 = _NBUF * j
            for k in range(_NBUF):
                step(i0 + k, k, (k + 1) % _NBUF)
            return carry

        lax.fori_loop(0, n_groups, body, 0)
        base = n_groups * _NBUF
        for k in range(n_tail):
            i = base + k
            p = i % _NBUF
            if k + 1 < n_tail:
                pn = (i + 1) % _NBUF
                write_wait(i - (_NBUF - 1), pn)
                pltpu.async_copy(tbl_hbm.at[idx_v.at[i + 1]], bufs[pn], gsems[pn])
            gather_wait(i, p)
            write_start(i, p)
        for k in range(_NBUF):
            i = n_chunks - _NBUF + k
            write_wait(i, i % _NBUF)

    return sc_gather


def kernel(x, tok_embed, pos_embed, ln_w, ln_b):
    if x.ndim <= 1:
        x = x.reshape(1, -1)
    batch, seq = x.shape
    vocab, d = tok_embed.shape
    tbl, idx = _build_pair_table_and_idx(x, tok_embed, pos_embed, ln_w, ln_b)
    n_rows = batch * (seq // 2)
    info = plsc.get_sparse_core_info()
    n_workers = info.num_cores * info.num_subcores
    out = _make_sc_gather(n_rows, 2 * d, n_workers)(
        tbl, idx.reshape(-1, _CHUNK))
    return out.reshape(batch, seq, d)


# speedup vs baseline: 2.6114x; 2.6114x over previous
import functools

import jax
import jax.numpy as jnp
from jax import lax
from jax.experimental import pallas as pl
from jax.experimental.pallas import tpu as pltpu
from jax.experimental.pallas import tpu_sc as plsc


_REPL = 8


def _table_idx_body(x_ref, tok_ref, pos_ref, w_ref, b_ref, tbl_ref, idx_ref):
    vocab, d = tok_ref.shape
    seq = pos_ref.shape[0]
    emb = tok_ref[:][:, None, :] + pos_ref[:][None, :, :]
    mean = jnp.mean(emb, axis=-1, keepdims=True)
    var = jnp.mean(jnp.square(emb - mean), axis=-1, keepdims=True)
    normed = (emb - mean) * lax.rsqrt(var + 1e-5)
    t = normed * w_ref[:][None, None, :] + b_ref[:][None, None, :]
    tbl_ref[...] = jnp.broadcast_to(
        t.reshape(1, vocab * seq, d), (_REPL, vocab * seq, d))
    b_iota = lax.broadcasted_iota(jnp.int32, x_ref.shape, 0)
    s_iota = lax.broadcasted_iota(jnp.int32, x_ref.shape, 1)
    flat = b_iota * seq + s_iota
    rep = lax.shift_right_logical(flat, 7) & (_REPL - 1)
    idx_ref[...] = x_ref[...] * seq + s_iota + rep * (vocab * seq)


def _build_table_and_idx(x, tok_embed, pos_embed, ln_w, ln_b):
    vocab, d = tok_embed.shape
    seq = x.shape[1]
    tbl, idx = pl.pallas_call(
        _table_idx_body,
        out_shape=[
            jax.ShapeDtypeStruct((_REPL, vocab * seq, d), jnp.float32),
            jax.ShapeDtypeStruct(x.shape, jnp.int32),
        ],
    )(x, tok_embed, pos_embed[:seq], ln_w, ln_b)
    return tbl.reshape(_REPL * vocab * seq, d), idx.reshape(-1)


_CHUNK = 128


_NBUF = 3


def _make_sc_gather(n_rows, vocab_seq, d, n_workers):
    rows_per_w = n_rows // n_workers
    n_chunks = rows_per_w // _CHUNK
    n_groups = (n_chunks - 1) // _NBUF
    n_tail = n_chunks - n_groups * _NBUF
    mesh = plsc.VectorSubcoreMesh(core_axis_name="c", subcore_axis_name="s")

    @functools.partial(
        pl.kernel,
        mesh=mesh,
        out_type=jax.ShapeDtypeStruct((n_rows, d), jnp.float32),
        scratch_types=[
            pltpu.VMEM((n_chunks, _CHUNK), jnp.int32),
            [pltpu.VMEM((_CHUNK, d), jnp.float32)] * _NBUF,
            [pltpu.SemaphoreType.DMA] * _NBUF,
            [pltpu.SemaphoreType.DMA] * _NBUF,
        ],
    )
    def sc_gather(tbl_hbm, idx_hbm, out_hbm, idx_v, bufs, gsems, wsems):
        n_cores = lax.axis_size("c")
        wid = lax.axis_index("s") * n_cores + lax.axis_index("c")
        cbase = wid * n_chunks
        pltpu.sync_copy(idx_hbm.at[pl.ds(cbase, n_chunks)], idx_v)
        pltpu.async_copy(tbl_hbm.at[idx_v.at[0]], bufs[0], gsems[0])

        def gather_wait(i, p):
            pltpu.make_async_copy(tbl_hbm.at[idx_v.at[i]], bufs[p], gsems[p]).wait()

        def write_start(i, p):
            pltpu.async_copy(
                bufs[p], out_hbm.at[pl.ds((cbase + i) * _CHUNK, _CHUNK)], wsems[p])

        def write_wait(i, p):
            pltpu.make_async_copy(
                bufs[p], out_hbm.at[pl.ds((cbase + i) * _CHUNK, _CHUNK)], wsems[p]
            ).wait()

        def step(i, p, pn):
            gather_wait(i, p)
            write_start(i, p)
            @pl.when(i >= _NBUF - 1)
            def _():
                write_wait(i - (_NBUF - 1), pn)

            pltpu.async_copy(tbl_hbm.at[idx_v.at[i + 1]], bufs[pn], gsems[pn])

        def body(j, carry):
            i0 = _NBUF * j
            for k in range(_NBUF):
                step(i0 + k, k, (k + 1) % _NBUF)
            return carry

        lax.fori_loop(0, n_groups, body, 0)
        base = n_groups * _NBUF
        for k in range(n_tail):
            i = base + k
            p = i % _NBUF
            if k + 1 < n_tail:
                pn = (i + 1) % _NBUF
                write_wait(i - (_NBUF - 1), pn)
                pltpu.async_copy(tbl_hbm.at[idx_v.at[i + 1]], bufs[pn], gsems[pn])
            gather_wait(i, p)
            write_start(i, p)
        for k in range(_NBUF):
            i = n_chunks - _NBUF + k
            write_wait(i, i % _NBUF)

    return sc_gather


def kernel(x, tok_embed, pos_embed, ln_w, ln_b):
    if x.ndim <= 1:
        x = x.reshape(1, -1)
    batch, seq = x.shape
    vocab, d = tok_embed.shape
    tbl, idx = _build_table_and_idx(x, tok_embed, pos_embed, ln_w, ln_b)
    n_rows = batch * seq
    info = plsc.get_sparse_core_info()
    n_workers = info.num_cores * info.num_subcores
    out = _make_sc_gather(n_rows, vocab * seq, d, n_workers)(
        tbl, idx.reshape(-1, _CHUNK))
    return out.reshape(batch, seq, d)
